# Initial kernel scaffold; baseline (speedup 1.0000x reference)
import jax
import jax.numpy as jnp
from jax.experimental import pallas as pl


def kernel(r, cos, idx, z, edge_emb, trip_emb, Wg_src, Wg_dst, Wg_e, Wg_bf, Wg_v, Wh_src, Wh_dst, Wh_e, Wh_bf, Wh_v, fc1_w, fc1_b, fc2_w, fc2_b):
    raise NotImplementedError("write your pallas kernel here")


# speedup vs baseline: 2.6227x; 2.6227x over previous
import functools

import jax
import jax.numpy as jnp
from jax import lax
from jax.experimental import pallas as pl
from jax.experimental.pallas import tpu as pltpu
from jax.experimental.pallas import tpu_sc as plsc

_K = 8
_HID = 32
_RAD = 32
_NC = 2
_NS = 16
_NW = _NC * _NS
_CW = 128


def _sc_gather(table, idx3):
    nw, ch, cw = idx3.shape
    d = table.shape[1]
    mesh = plsc.VectorSubcoreMesh(core_axis_name="c", subcore_axis_name="s")

    @functools.partial(
        pl.kernel,
        out_type=jax.ShapeDtypeStruct((nw, ch, cw, d), table.dtype),
        mesh=mesh,
        compiler_params=pltpu.CompilerParams(use_tc_tiling_on_sc=False),
        scratch_types=[
            pltpu.VMEM((ch, cw), jnp.int32),
            pltpu.VMEM((ch, cw, d), table.dtype),
            pltpu.SemaphoreType.DMA,
        ],
    )
    def gk(table_hbm, idx_hbm, out_hbm, idx_v, rows_v, sem):
        wid = lax.axis_index("s") * _NC + lax.axis_index("c")
        pltpu.sync_copy(idx_hbm.at[wid], idx_v)
        copies = [
            pltpu.async_copy(table_hbm.at[idx_v.at[j]], rows_v.at[j], sem)
            for j in range(ch)
        ]
        for c in copies:
            c.wait()
        pltpu.sync_copy(rows_v, out_hbm.at[wid])

    return gk(table, idx3)


def _mm(a, w):
    return jnp.dot(a.astype(jnp.bfloat16), w.astype(jnp.bfloat16),
                   preferred_element_type=jnp.float32)


def _rbf(x, lo, hi):
    step = (hi - lo) / (_RAD - 1)
    c = lo + step * lax.broadcasted_iota(
        jnp.int32, x.shape[:-1] + (_RAD,), x.ndim - 1).astype(jnp.float32)
    g = ((_RAD - 1) / (hi - lo)) ** 2
    return jnp.exp(-g * (x - c) ** 2)


def _cutoff(rr):
    u = jnp.clip((rr - 0.95) / 0.05, 0.0, 1.0)
    return 0.5 * (jnp.cos(jnp.pi * u) + 1.0)


def _sigmoid(x):
    return 1.0 / (1.0 + jnp.exp(-x))


def _silu(x):
    return x * _sigmoid(x)


def _tidx_onehot(zb, zjb, bn):
    eq_ij = (zb == zjb).astype(jnp.int32)
    eq_jk = (zjb[:, :, None] == zjb[:, None, :]).astype(jnp.int32)
    t = eq_ij[:, :, None] * 4 + eq_ij[:, None, :] * 2 + eq_jk
    io = lax.broadcasted_iota(jnp.int32, (bn, _K, _K, 8), 3)
    t_oh = (t[..., None] == io).astype(jnp.float32).reshape(bn * _K * _K, 8)
    return eq_ij, t_oh


def _hconv(he, cosb, t_oh, cut_h, tr, whsrc, whdst, whe, whbf, whv, bn):
    te_tab = _mm(tr, whe)
    he2 = he.reshape(bn * _K, _HID)
    ms = _mm(he2, whsrc).reshape(bn, _K, _HID)
    md = _mm(he2, whdst).reshape(bn, _K, _HID)
    vh = _mm(he2, whv).reshape(bn, _K, _HID)
    feats = _rbf(cosb[..., None], -1.0, 1.0)
    filt = _mm(feats.reshape(bn * _K * _K, _RAD), whbf).reshape(bn, _K, _K, _HID)
    te = _mm(t_oh, te_tab).reshape(bn, _K, _K, _HID)
    gate = _sigmoid(ms[:, :, None, :] + md[:, None, :, :] + te)
    prod = gate * filt * vh[:, None, :, :] * cut_h[..., None]
    agg = jnp.sum(prod, axis=2)
    return _silu(agg)


def _l0_body(r_ref, cos_ref, z_ref, zj_ref, ee_ref, tr_ref,
             wgsrc, wgdst, wge, wgbf, wgv,
             whsrc, whdst, whe, whbf, whv,
             hn_out, he_out):
    bn = r_ref.shape[0]
    rb = r_ref[...]
    cosb = cos_ref[...]
    eq_ij, t_oh = _tidx_onehot(z_ref[...], zj_ref[...], bn)
    cut_g = _cutoff(rb)
    cut_h = jnp.minimum(cut_g[:, :, None], cut_g[:, None, :])
    ee = ee_ref[...]
    eqf = eq_ij.astype(jnp.float32)
    he0 = jnp.where(eqf[..., None] > 0.5,
                    ee[1][None, None, :], ee[0][None, None, :])
    y = _hconv(he0, cosb, t_oh, cut_h, tr_ref[...],
               whsrc[...], whdst[...], whe[...], whbf[...], whv[...], bn)
    he_out[...] = he0 + y
    cs_src = jnp.sum(wgsrc[...], axis=0)
    cs_dst = jnp.sum(wgdst[...], axis=0)
    cs_v = jnp.sum(wgv[...], axis=0)
    ge = _mm(he0.reshape(bn * _K, _HID), wge[...]).reshape(bn, _K, _HID)
    gate = _sigmoid(cs_src[None, None, :] + cs_dst[None, None, :] + ge)
    bf_g = _rbf(rb[..., None], 0.0, 1.0)
    filt = _mm(bf_g.reshape(bn * _K, _RAD), wgbf[...]).reshape(bn, _K, _HID)
    agg = jnp.sum(gate * filt * cs_v[None, None, :] * cut_g[..., None], axis=1)
    hn_out[...] = 1.0 + _silu(agg)


def _hlayer_body(r_ref, cos_ref, z_ref, zj_ref, tr_ref,
                 whsrc, whdst, whe, whbf, whv, he_ref, he_out):
    bn = r_ref.shape[0]
    rb = r_ref[...]
    _, t_oh = _tidx_onehot(z_ref[...], zj_ref[...], bn)
    cut_g = _cutoff(rb)
    cut_h = jnp.minimum(cut_g[:, :, None], cut_g[:, None, :])
    he = he_ref[...]
    y = _hconv(he, cos_ref[...], t_oh, cut_h, tr_ref[...],
               whsrc[...], whdst[...], whe[...], whbf[...], whv[...], bn)
    he_out[...] = he + y


def _gconv(rb, he, hn, xj, wgsrc, wgdst, wge, wgbf, wgv, bn):
    cut_g = _cutoff(rb)
    sg = _mm(hn, wgsrc)
    gd = _mm(xj, wgdst).reshape(bn, _K, _HID)
    ge = _mm(he.reshape(bn * _K, _HID), wge).reshape(bn, _K, _HID)
    vg = _mm(xj, wgv).reshape(bn, _K, _HID)
    gate = _sigmoid(sg[:, None, :] + gd + ge)
    bf_g = _rbf(rb[..., None], 0.0, 1.0)
    filt = _mm(bf_g.reshape(bn * _K, _RAD), wgbf).reshape(bn, _K, _HID)
    agg = jnp.sum(gate * filt * vg * cut_g[..., None], axis=1)
    return hn + _silu(agg)


def _glayer_body(r_ref, he_ref, hn_ref, xj_ref,
                 wgsrc, wgdst, wge, wgbf, wgv, hn_out):
    bn = r_ref.shape[0]
    hn_out[...] = _gconv(r_ref[...], he_ref[...], hn_ref[...], xj_ref[...],
                         wgsrc[...], wgdst[...], wge[...], wgbf[...], wgv[...],
                         bn)


def _gfinal_body(r_ref, he_ref, hn_ref, xj_ref,
                 wgsrc, wgdst, wge, wgbf, wgv,
                 f1w, f1b, f2w, f2b, acc_out, *, n_total):
    i = pl.program_id(0)
    bn = r_ref.shape[0]
    hn3 = _gconv(r_ref[...], he_ref[...], hn_ref[...], xj_ref[...],
                 wgsrc[...], wgdst[...], wge[...], wgbf[...], wgv[...], bn)
    x = _silu(_mm(hn3, f1w[...]) + f1b[...])
    x2 = _silu(_mm(x, f2w[...]) + f2b[...])
    part = jnp.sum(x2, axis=0, keepdims=True) * (1.0 / n_total)

    @pl.when(i == 0)
    def _():
        acc_out[...] = jnp.zeros_like(acc_out)

    acc_out[...] += part


def kernel(r, cos, idx, z, edge_emb, trip_emb,
           Wg_src, Wg_dst, Wg_e, Wg_bf, Wg_v,
           Wh_src, Wh_dst, Wh_e, Wh_bf, Wh_v,
           fc1_w, fc1_b, fc2_w, fc2_b):
    n, k = r.shape
    bn = 200
    g = n // bn
    out_dim = fc2_w.shape[1]

    nflat = n * k
    ch = -(-nflat // (_NW * _CW))
    ipad = _NW * ch * _CW
    idx3 = jnp.pad(idx.reshape(-1), (0, ipad - nflat)).reshape(_NW, ch, _CW)

    z16 = jnp.broadcast_to(z[:, None], (n, 16))
    zj = _sc_gather(z16, idx3).reshape(ipad, 16)[:nflat, 0].reshape(n, k)
    z2 = z[:, None]

    spec_r = pl.BlockSpec((bn, k), lambda i: (i, 0))
    spec_cos = pl.BlockSpec((bn, k, k), lambda i: (i, 0, 0))
    spec_z = pl.BlockSpec((bn, 1), lambda i: (i, 0))
    spec_he = pl.BlockSpec((bn, k, _HID), lambda i: (i, 0, 0))
    spec_hn = pl.BlockSpec((bn, _HID), lambda i: (i, 0))
    spec_xj = pl.BlockSpec((bn * k, _HID), lambda i: (i, 0))

    def wspec(a):
        nd = a.ndim
        return pl.BlockSpec(a.shape, lambda i, _nd=nd: (0,) * _nd)

    hn1, he1 = pl.pallas_call(
        _l0_body,
        grid=(g,),
        in_specs=[spec_r, spec_cos, spec_z, spec_r,
                  wspec(edge_emb), wspec(trip_emb),
                  wspec(Wg_src[0]), wspec(Wg_dst[0]), wspec(Wg_e[0]),
                  wspec(Wg_bf[0]), wspec(Wg_v[0]),
                  wspec(Wh_src[0]), wspec(Wh_dst[0]), wspec(Wh_e[0]),
                  wspec(Wh_bf[0]), wspec(Wh_v[0])],
        out_specs=[spec_hn, spec_he],
        out_shape=[jax.ShapeDtypeStruct((n, _HID), jnp.float32),
                   jax.ShapeDtypeStruct((n, k, _HID), jnp.float32)],
    )(r, cos, z2, zj, edge_emb, trip_emb,
      Wg_src[0], Wg_dst[0], Wg_e[0], Wg_bf[0], Wg_v[0],
      Wh_src[0], Wh_dst[0], Wh_e[0], Wh_bf[0], Wh_v[0])

    xj1 = _sc_gather(hn1, idx3).reshape(ipad, _HID)

    he2 = pl.pallas_call(
        _hlayer_body,
        grid=(g,),
        in_specs=[spec_r, spec_cos, spec_z, spec_r, wspec(trip_emb),
                  wspec(Wh_src[1]), wspec(Wh_dst[1]), wspec(Wh_e[1]),
                  wspec(Wh_bf[1]), wspec(Wh_v[1]), spec_he],
        out_specs=spec_he,
        out_shape=jax.ShapeDtypeStruct((n, k, _HID), jnp.float32),
    )(r, cos, z2, zj, trip_emb,
      Wh_src[1], Wh_dst[1], Wh_e[1], Wh_bf[1], Wh_v[1], he1)

    hn2 = pl.pallas_call(
        _glayer_body,
        grid=(g,),
        in_specs=[spec_r, spec_he, spec_hn, spec_xj,
                  wspec(Wg_src[1]), wspec(Wg_dst[1]), wspec(Wg_e[1]),
                  wspec(Wg_bf[1]), wspec(Wg_v[1])],
        out_specs=spec_hn,
        out_shape=jax.ShapeDtypeStruct((n, _HID), jnp.float32),
    )(r, he1, hn1, xj1,
      Wg_src[1], Wg_dst[1], Wg_e[1], Wg_bf[1], Wg_v[1])

    xj2 = _sc_gather(hn2, idx3).reshape(ipad, _HID)

    acc = pl.pallas_call(
        functools.partial(_gfinal_body, n_total=float(n)),
        grid=(g,),
        in_specs=[spec_r, spec_he, spec_hn, spec_xj,
                  wspec(Wg_src[2]), wspec(Wg_dst[2]), wspec(Wg_e[2]),
                  wspec(Wg_bf[2]), wspec(Wg_v[2]),
                  wspec(fc1_w), pl.BlockSpec((1, _HID), lambda i: (0, 0)),
                  wspec(fc2_w), pl.BlockSpec((1, out_dim), lambda i: (0, 0))],
        out_specs=pl.BlockSpec((1, out_dim), lambda i: (0, 0)),
        out_shape=jax.ShapeDtypeStruct((1, out_dim), jnp.float32),
    )(r, he2, hn2, xj2,
      Wg_src[2], Wg_dst[2], Wg_e[2], Wg_bf[2], Wg_v[2],
      fc1_w, fc1_b[None, :], fc2_w, fc2_b[None, :])

    return acc[0]


# speedup vs baseline: 3.5701x; 1.3612x over previous
import functools

import jax
import jax.numpy as jnp
from jax import lax
from jax.experimental import pallas as pl
from jax.experimental.pallas import tpu as pltpu
from jax.experimental.pallas import tpu_sc as plsc

_K = 8
_HID = 32
_RAD = 32
_KH = _K * _HID
_NC = 2
_NS = 16
_NW = _NC * _NS
_CW = 128

_GAMMA_G = float((_RAD - 1) ** 2)
_GAMMA_H = float(((_RAD - 1) / 2.0) ** 2)


def _sc_gather(table, idx3):
    nw, ch, cw = idx3.shape
    d = table.shape[1]
    mesh = plsc.VectorSubcoreMesh(core_axis_name="c", subcore_axis_name="s")

    @functools.partial(
        pl.kernel,
        out_type=jax.ShapeDtypeStruct((nw, ch, cw, d), table.dtype),
        mesh=mesh,
        compiler_params=pltpu.CompilerParams(use_tc_tiling_on_sc=False),
        scratch_types=[
            pltpu.VMEM((ch, cw), jnp.int32),
            pltpu.VMEM((ch, cw, d), table.dtype),
            pltpu.SemaphoreType.DMA,
        ],
    )
    def gk(table_hbm, idx_hbm, out_hbm, idx_v, rows_v, sem):
        wid = lax.axis_index("s") * _NC + lax.axis_index("c")
        pltpu.sync_copy(idx_hbm.at[wid], idx_v)
        copies = [
            pltpu.async_copy(table_hbm.at[idx_v.at[j]], rows_v.at[j], sem)
            for j in range(ch)
        ]
        for c in copies:
            c.wait()
        pltpu.sync_copy(rows_v, out_hbm.at[wid])

    return gk(table, idx3)


def _mm(a, w):
    return jnp.dot(a.astype(jnp.bfloat16), w.astype(jnp.bfloat16),
                   preferred_element_type=jnp.float32)


def _sigmoid(x):
    return 1.0 / (1.0 + jnp.exp(-x))


def _silu(x):
    return x * _sigmoid(x)


def _cutoff(rr):
    u = jnp.clip((rr - 0.95) / 0.05, 0.0, 1.0)
    return 0.5 * (jnp.cos(jnp.pi * u) + 1.0)


def _iota2(shape, dim):
    return lax.broadcasted_iota(jnp.int32, shape, dim)


def _rowrep8(x):
    m, l = x.shape
    return jnp.broadcast_to(x[:, None, :], (m, 8, l)).reshape(m * 8, l)


def _consts():
    t32 = (_iota2((_K, _KH), 1) // _HID == _iota2((_K, _KH), 0)
           ).astype(jnp.float32)
    t8 = (_iota2((_K, 64), 1) // 8 == _iota2((_K, 64), 0)).astype(jnp.float32)
    a64 = (_iota2((64, 8), 0) % 8 == _iota2((64, 8), 1)).astype(jnp.float32)
    ti = (_iota2((_HID, _KH), 1) % _HID == _iota2((_HID, _KH), 0)
          ).astype(jnp.float32)
    summ = (_iota2((_KH, _HID), 0) % _HID == _iota2((_KH, _HID), 1)
            ).astype(jnp.float32)
    mask64 = (_iota2((64, _KH), 0) // 8 == _iota2((64, _KH), 1) // _HID
              ).astype(jnp.float32)
    return t32, t8, a64, ti, summ, mask64


def _cen256(lo, hi):
    step = (hi - lo) / (_RAD - 1)
    return lo + step * (_iota2((1, _KH), 1) % _RAD).astype(jnp.float32)


def _hconv256(he_rows, he256, cos_rows, z2, zj8, zrow, zjrow, rb8, r_row,
              trip, whe, wsrcT, bdd, bdv, bdbf, bn):
    t32, t8, a64, ti, summ, mask64 = _consts()
    r2 = bn * _K
    cg8 = _cutoff(rb8)
    cg_lane = _mm(_rowrep8(cg8), t32)
    cut256 = jnp.minimum(cg_lane, _cutoff(r_row))
    eq_row = (zrow == zjrow).astype(jnp.float32)
    eq_lane = _rowrep8((z2 == zj8).astype(jnp.float32))
    eq_jk = (zjrow == _rowrep8(zj8)).astype(jnp.float32)
    t8v = eq_row * 4.0 + eq_lane * 2.0 + eq_jk
    t_rep = _mm(t8v, t8)
    c64 = (_iota2((1, 64), 1) % 8).astype(jnp.float32)
    oh64 = (t_rep == c64).astype(jnp.float32)
    te_tab = _mm(trip, whe)
    te_bd = _mm(_mm(a64, te_tab), ti) * mask64
    te256 = _mm(oh64, te_bd)
    ms_t = _mm(he_rows, wsrcT)
    md256 = _rowrep8(_mm(he256, bdd))
    vh256 = _rowrep8(_mm(he256, bdv))
    cos_rep = _mm(cos_rows, t32)
    feats = jnp.exp(-_GAMMA_H * (cos_rep - _cen256(-1.0, 1.0)) ** 2)
    filt = _mm(feats, bdbf)
    gate = _sigmoid(ms_t + md256 + te256)
    prod = gate * filt * vh256 * cut256
    agg = _mm(prod, summ)
    return _silu(agg)


def _gconv256(rb8, he256, hn, xj256, wsrcT, bdd, bde, bdv, bdbf, bn):
    t32, _, _, _, summ, _ = _consts()
    cg8 = _cutoff(rb8)
    cut256 = _mm(cg8, t32)
    sg_t = _mm(hn, wsrcT)
    gate = _sigmoid(sg_t + _mm(xj256, bdd) + _mm(he256, bde))
    r_rep = _mm(rb8, t32)
    feats = jnp.exp(-_GAMMA_G * (r_rep - _cen256(0.0, 1.0)) ** 2)
    filt = _mm(feats, bdbf)
    vg = _mm(xj256, bdv)
    prod = gate * filt * vg * cut256
    agg = _mm(prod, summ)
    return hn + _silu(agg)


def _l0_body(rb_ref, rrow_ref, cosr_ref, z2_ref, zj8_ref, zrow_ref, zjrow_ref,
             ee_ref, tr_ref, whe, wsrcT, bdd, bdv, bdbf,
             wgsrc, wgdst, wgv, bdge, bdgbf,
             hn_out, he_out):
    bn = rb_ref.shape[0]
    r2 = bn * _K
    t32, _, _, ti, summ, _ = _consts()
    z2 = z2_ref[...]
    zj8 = zj8_ref[...]
    zrow = zrow_ref[...]
    zjrow = zjrow_ref[...]
    ee = ee_ref[...]
    eq_row = (zrow == zjrow).astype(jnp.float32)
    ee0t = _mm(ee[0:1, :], ti)
    ee1t = _mm(ee[1:2, :], ti)
    he_rows = (eq_row * ee[1:2, :].astype(jnp.float32)
               + (1.0 - eq_row) * ee[0:1, :].astype(jnp.float32))
    eq256 = _mm((z2 == zj8).astype(jnp.float32), t32)
    he256 = eq256 * ee1t + (1.0 - eq256) * ee0t
    y = _hconv256(he_rows, he256, cosr_ref[...], z2, zj8, zrow, zjrow,
                  rb_ref[...], rrow_ref[...],
                  tr_ref[...], whe[...], wsrcT[...], bdd[...], bdv[...],
                  bdbf[...], bn)
    he_out[...] = he_rows + y
    rb8 = rb_ref[...]
    cg8 = _cutoff(rb8)
    cut256 = _mm(cg8, t32)
    cs_src_t = _mm(jnp.sum(wgsrc[...], axis=0, keepdims=True), ti)
    cs_dst_t = _mm(jnp.sum(wgdst[...], axis=0, keepdims=True), ti)
    cs_v_t = _mm(jnp.sum(wgv[...], axis=0, keepdims=True), ti)
    gate = _sigmoid(cs_src_t + cs_dst_t + _mm(he256, bdge[...]))
    r_rep = _mm(rb8, t32)
    feats = jnp.exp(-_GAMMA_G * (r_rep - _cen256(0.0, 1.0)) ** 2)
    filt = _mm(feats, bdgbf[...])
    prod = gate * filt * cs_v_t * cut256
    agg = _mm(prod, summ)
    hn_out[...] = 1.0 + _silu(agg)


def _hlayer_body(rb_ref, rrow_ref, cosr_ref, z2_ref, zj8_ref, zrow_ref,
                 zjrow_ref, tr_ref, whe, wsrcT, bdd, bdv, bdbf,
                 her_ref, he256_ref, he_out):
    bn = rb_ref.shape[0]
    he_rows = her_ref[...]
    y = _hconv256(he_rows, he256_ref[...], cosr_ref[...], z2_ref[...],
                  zj8_ref[...], zrow_ref[...], zjrow_ref[...],
                  rb_ref[...], rrow_ref[...],
                  tr_ref[...], whe[...], wsrcT[...], bdd[...], bdv[...],
                  bdbf[...], bn)
    he_out[...] = he_rows + y


def _glayer_body(rb_ref, he256_ref, hn_ref, xj_ref,
                 wsrcT, bdd, bde, bdv, bdbf, hn_out):
    bn = rb_ref.shape[0]
    hn_out[...] = _gconv256(rb_ref[...], he256_ref[...], hn_ref[...],
                            xj_ref[...], wsrcT[...], bdd[...], bde[...],
                            bdv[...], bdbf[...], bn)


def _gfinal_body(rb_ref, he256_ref, hn_ref, xj_ref,
                 wsrcT, bdd, bde, bdv, bdbf,
                 f1w, f1b, f2w, f2b, acc_out, *, n_total):
    i = pl.program_id(0)
    bn = rb_ref.shape[0]
    hn3 = _gconv256(rb_ref[...], he256_ref[...], hn_ref[...], xj_ref[...],
                    wsrcT[...], bdd[...], bde[...], bdv[...], bdbf[...], bn)
    x = _silu(_mm(hn3, f1w[...]) + f1b[...])
    x2 = _silu(_mm(x, f2w[...]) + f2b[...])
    part = jnp.sum(x2, axis=0, keepdims=True) * (1.0 / n_total)

    @pl.when(i == 0)
    def _():
        acc_out[...] = jnp.zeros_like(acc_out)

    acc_out[...] += part


def kernel(r, cos, idx, z, edge_emb, trip_emb,
           Wg_src, Wg_dst, Wg_e, Wg_bf, Wg_v,
           Wh_src, Wh_dst, Wh_e, Wh_bf, Wh_v,
           fc1_w, fc1_b, fc2_w, fc2_b):
    n, k = r.shape
    bn = 200
    g = n // bn
    bng = 1000 if n % 1000 == 0 else bn
    gg = n // bng
    out_dim = fc2_w.shape[1]
    f32 = jnp.float32

    nflat = n * k
    ch = -(-nflat // (_NW * _CW))
    ipad = _NW * ch * _CW
    idx3 = jnp.pad(idx.reshape(-1), (0, ipad - nflat)).reshape(_NW, ch, _CW)

    z16 = jnp.broadcast_to(z[:, None], (n, 16))
    zj = _sc_gather(z16, idx3).reshape(ipad, 16)[:nflat, 0].reshape(n, k)

    z2 = z[:, None]
    zrow = jnp.broadcast_to(z[:, None], (n, k)).reshape(nflat, 1)
    zjrow = zj.reshape(nflat, 1)
    r_row = r.reshape(nflat, 1)
    cos_rows = cos.reshape(nflat, k)

    eye8 = jnp.eye(8, dtype=f32)
    bd = lambda w: jnp.kron(eye8, w.astype(f32))
    tile8 = lambda w: jnp.tile(w.astype(f32), (1, 8))

    spec_rb = pl.BlockSpec((bn, k), lambda i: (i, 0))
    spec_row = pl.BlockSpec((bn * k, 1), lambda i: (i, 0))
    spec_cosr = pl.BlockSpec((bn * k, k), lambda i: (i, 0))
    spec_z2 = pl.BlockSpec((bn, 1), lambda i: (i, 0))
    spec_her = pl.BlockSpec((bn * k, _HID), lambda i: (i, 0))
    spec_he256 = pl.BlockSpec((bn, _KH), lambda i: (i, 0))
    spec_hn = pl.BlockSpec((bn, _HID), lambda i: (i, 0))

    def wspec(a):
        nd = a.ndim
        return pl.BlockSpec(a.shape, lambda i, _nd=nd: (0,) * _nd)

    bdv_h0 = bd(Wh_v[0])
    hn1, he1 = pl.pallas_call(
        _l0_body,
        grid=(g,),
        in_specs=[spec_rb, spec_row, spec_cosr, spec_z2, spec_rb,
                  spec_row, spec_row,
                  wspec(edge_emb), wspec(trip_emb)] + [wspec(edge_emb)] * 0 +
                 [pl.BlockSpec((_HID, _HID), lambda i: (0, 0)),
                  pl.BlockSpec((_HID, _KH), lambda i: (0, 0)),
                  pl.BlockSpec((_KH, _KH), lambda i: (0, 0)),
                  pl.BlockSpec((_KH, _KH), lambda i: (0, 0)),
                  pl.BlockSpec((_KH, _KH), lambda i: (0, 0)),
                  pl.BlockSpec((_HID, _HID), lambda i: (0, 0)),
                  pl.BlockSpec((_HID, _HID), lambda i: (0, 0)),
                  pl.BlockSpec((_HID, _HID), lambda i: (0, 0)),
                  pl.BlockSpec((_KH, _KH), lambda i: (0, 0)),
                  pl.BlockSpec((_KH, _KH), lambda i: (0, 0))],
        out_specs=[spec_hn, spec_her],
        out_shape=[jax.ShapeDtypeStruct((n, _HID), f32),
                   jax.ShapeDtypeStruct((nflat, _HID), f32)],
    )(r, r_row, cos_rows, z2, zj, zrow, zjrow, edge_emb, trip_emb,
      Wh_e[0], tile8(Wh_src[0]), bd(Wh_dst[0]), bdv_h0, bd(Wh_bf[0]),
      Wg_src[0], Wg_dst[0], Wg_v[0], bd(Wg_e[0]), bd(Wg_bf[0]))

    xj1 = _sc_gather(hn1, idx3).reshape(ipad // k, _KH)

    he1_256 = he1.reshape(n, _KH)
    he2 = pl.pallas_call(
        _hlayer_body,
        grid=(g,),
        in_specs=[spec_rb, spec_row, spec_cosr, spec_z2, spec_rb,
                  spec_row, spec_row, wspec(trip_emb),
                  pl.BlockSpec((_HID, _HID), lambda i: (0, 0)),
                  pl.BlockSpec((_HID, _KH), lambda i: (0, 0)),
                  pl.BlockSpec((_KH, _KH), lambda i: (0, 0)),
                  pl.BlockSpec((_KH, _KH), lambda i: (0, 0)),
                  pl.BlockSpec((_KH, _KH), lambda i: (0, 0)),
                  spec_her, spec_he256],
        out_specs=spec_her,
        out_shape=jax.ShapeDtypeStruct((nflat, _HID), f32),
    )(r, r_row, cos_rows, z2, zj, zrow, zjrow, trip_emb,
      Wh_e[1], tile8(Wh_src[1]), bd(Wh_dst[1]), bd(Wh_v[1]), bd(Wh_bf[1]),
      he1, he1_256)

    spec_rbg = pl.BlockSpec((bng, k), lambda i: (i, 0))
    spec_he256g = pl.BlockSpec((bng, _KH), lambda i: (i, 0))
    spec_hng = pl.BlockSpec((bng, _HID), lambda i: (i, 0))

    hn2 = pl.pallas_call(
        _glayer_body,
        grid=(gg,),
        in_specs=[spec_rbg, spec_he256g, spec_hng, spec_he256g,
                  pl.BlockSpec((_HID, _KH), lambda i: (0, 0)),
                  pl.BlockSpec((_KH, _KH), lambda i: (0, 0)),
                  pl.BlockSpec((_KH, _KH), lambda i: (0, 0)),
                  pl.BlockSpec((_KH, _KH), lambda i: (0, 0)),
                  pl.BlockSpec((_KH, _KH), lambda i: (0, 0))],
        out_specs=spec_hng,
        out_shape=jax.ShapeDtypeStruct((n, _HID), f32),
    )(r, he1_256, hn1, xj1,
      tile8(Wg_src[1]), bd(Wg_dst[1]), bd(Wg_e[1]), bd(Wg_v[1]),
      bd(Wg_bf[1]))

    xj2 = _sc_gather(hn2, idx3).reshape(ipad // k, _KH)

    acc = pl.pallas_call(
        functools.partial(_gfinal_body, n_total=float(n)),
        grid=(gg,),
        in_specs=[spec_rbg, spec_he256g, spec_hng, spec_he256g,
                  pl.BlockSpec((_HID, _KH), lambda i: (0, 0)),
                  pl.BlockSpec((_KH, _KH), lambda i: (0, 0)),
                  pl.BlockSpec((_KH, _KH), lambda i: (0, 0)),
                  pl.BlockSpec((_KH, _KH), lambda i: (0, 0)),
                  pl.BlockSpec((_KH, _KH), lambda i: (0, 0)),
                  wspec(fc1_w), pl.BlockSpec((1, _HID), lambda i: (0, 0)),
                  wspec(fc2_w), pl.BlockSpec((1, out_dim), lambda i: (0, 0))],
        out_specs=pl.BlockSpec((1, out_dim), lambda i: (0, 0)),
        out_shape=jax.ShapeDtypeStruct((1, out_dim), f32),
    )(r, he2.reshape(n, _KH), hn2, xj2,
      tile8(Wg_src[2]), bd(Wg_dst[2]), bd(Wg_e[2]), bd(Wg_v[2]),
      bd(Wg_bf[2]),
      fc1_w, fc1_b[None, :], fc2_w, fc2_b[None, :])

    return acc[0]


# speedup vs baseline: 4.6691x; 1.3078x over previous
import functools

import jax
import jax.numpy as jnp
from jax import lax
from jax.experimental import pallas as pl
from jax.experimental.pallas import tpu as pltpu
from jax.experimental.pallas import tpu_sc as plsc

_K = 8
_HID = 32
_RAD = 32
_KH = _K * _HID
_NC = 2
_NS = 16
_NW = _NC * _NS
_CW = 128

_GAMMA_G = float((_RAD - 1) ** 2)
_GAMMA_H = float(((_RAD - 1) / 2.0) ** 2)


def _sc_gather(table, idx3):
    nw, ch, cw = idx3.shape
    d = table.shape[1]
    mesh = plsc.VectorSubcoreMesh(core_axis_name="c", subcore_axis_name="s")

    @functools.partial(
        pl.kernel,
        out_type=jax.ShapeDtypeStruct((nw, ch, cw, d), table.dtype),
        mesh=mesh,
        compiler_params=pltpu.CompilerParams(use_tc_tiling_on_sc=False),
        scratch_types=[
            pltpu.VMEM((ch, cw), jnp.int32),
            pltpu.VMEM((ch, cw, d), table.dtype),
            pltpu.SemaphoreType.DMA,
        ],
    )
    def gk(table_hbm, idx_hbm, out_hbm, idx_v, rows_v, sem):
        wid = lax.axis_index("s") * _NC + lax.axis_index("c")
        pltpu.sync_copy(idx_hbm.at[wid], idx_v)
        copies = [
            pltpu.async_copy(table_hbm.at[idx_v.at[j]], rows_v.at[j], sem)
            for j in range(ch)
        ]
        for j, c in enumerate(copies):
            c.wait()
            pltpu.sync_copy(rows_v.at[j], out_hbm.at[wid, j])

    return gk(table, idx3)


def _mm(a, w):
    return jnp.dot(a.astype(jnp.bfloat16), w.astype(jnp.bfloat16),
                   preferred_element_type=jnp.float32)


def _sigmoid(x):
    return 1.0 / (1.0 + jnp.exp(-x))


def _silu(x):
    return x * _sigmoid(x)


def _cutoff(rr):
    u = jnp.clip((rr - 0.95) / 0.05, 0.0, 1.0)
    return 0.5 * (jnp.cos(jnp.pi * u) + 1.0)


def _iota2(shape, dim):
    return lax.broadcasted_iota(jnp.int32, shape, dim)


def _vtile8(x):
    m, l = x.shape
    return jnp.broadcast_to(x[None], (8, m, l)).reshape(8 * m, l)


def _fold256(x_rows, m):
    return jnp.concatenate([x_rows[j * m:(j + 1) * m, :] for j in range(8)],
                           axis=1)


def _consts():
    t32 = (_iota2((_K, _KH), 1) // _HID == _iota2((_K, _KH), 0)
           ).astype(jnp.float32)
    t8 = (_iota2((_K, 64), 1) // 8 == _iota2((_K, 64), 0)).astype(jnp.float32)
    a64 = (_iota2((64, 8), 0) % 8 == _iota2((64, 8), 1)).astype(jnp.float32)
    ti = (_iota2((_HID, _KH), 1) % _HID == _iota2((_HID, _KH), 0)
          ).astype(jnp.float32)
    summ = (_iota2((_KH, _HID), 0) % _HID == _iota2((_KH, _HID), 1)
            ).astype(jnp.float32)
    mask64 = (_iota2((64, _KH), 0) // 8 == _iota2((64, _KH), 1) // _HID
              ).astype(jnp.float32)
    return t32, t8, a64, ti, summ, mask64


def _cen256(lo, hi):
    step = (hi - lo) / (_RAD - 1)
    return lo + step * (_iota2((1, _KH), 1) % _RAD).astype(jnp.float32)


def _hshared(cos_rows, z2, zj8, zrowT, zjrowT, rb8, r_rowT):
    t32, t8, _, _, _, _ = _consts()
    cg8 = _cutoff(rb8)
    cg_lane = _vtile8(_mm(cg8, t32))
    cut256 = jnp.minimum(cg_lane, _cutoff(r_rowT))
    eq_row = (zrowT == zjrowT).astype(jnp.float32)
    eq_lane = _vtile8((z2 == zj8).astype(jnp.float32))
    eq_jk = (zjrowT == _vtile8(zj8)).astype(jnp.float32)
    t8v = eq_row * 4.0 + eq_lane * 2.0 + eq_jk
    t_rep = _mm(t8v, t8)
    c64 = (_iota2((1, 64), 1) % 8).astype(jnp.float32)
    oh64 = (t_rep == c64).astype(jnp.float32)
    cos_rep = _mm(cos_rows, t32)
    feats = jnp.exp(-_GAMMA_H * (cos_rep - _cen256(-1.0, 1.0)) ** 2)
    return eq_row, oh64, cut256, feats


def _hlayer(he_rows, he256, oh64, cut256, feats,
            trip, whe, wsrcT, bdd, bdv, bdbf):
    _, _, a64, ti, summ, mask64 = _consts()
    te_tab = _mm(trip, whe)
    te_bd = _mm(_mm(a64, te_tab), ti) * mask64
    te256 = _mm(oh64, te_bd)
    ms_t = _mm(he_rows, wsrcT)
    md256 = _vtile8(_mm(he256, bdd))
    vh256 = _vtile8(_mm(he256, bdv))
    filt = _mm(feats, bdbf)
    gate = _sigmoid(ms_t + md256 + te256)
    prod = gate * filt * vh256 * cut256
    agg = _mm(prod, summ)
    return _silu(agg)


def _gconv256(rb8, he256, hn, xj256, wsrcT, bdd, bde, bdv, bdbf, bn):
    t32, _, _, _, summ, _ = _consts()
    cg8 = _cutoff(rb8)
    cut256 = _mm(cg8, t32)
    sg_t = _mm(hn, wsrcT)
    gate = _sigmoid(sg_t + _mm(xj256, bdd) + _mm(he256, bde))
    r_rep = _mm(rb8, t32)
    feats = jnp.exp(-_GAMMA_G * (r_rep - _cen256(0.0, 1.0)) ** 2)
    filt = _mm(feats, bdbf)
    vg = _mm(xj256, bdv)
    prod = gate * filt * vg * cut256
    agg = _mm(prod, summ)
    return hn + _silu(agg)


def _he0_256(z2, zj8, ee):
    t32, _, _, ti, _, _ = _consts()
    eq256 = _mm((z2 == zj8).astype(jnp.float32), t32)
    return eq256 * _mm(ee[1:2, :], ti) + (1.0 - eq256) * _mm(ee[0:1, :], ti)


def _l0g_body(rb_ref, z2_ref, zj8_ref, ee_ref, wgsrc, wgdst, wgv, bdge, bdgbf,
              hn_out):
    t32, _, _, ti, summ, _ = _consts()
    rb8 = rb_ref[...]
    he256 = _he0_256(z2_ref[...], zj8_ref[...], ee_ref[...])
    cg8 = _cutoff(rb8)
    cut256 = _mm(cg8, t32)
    cs_src_t = _mm(jnp.sum(wgsrc[...], axis=0, keepdims=True), ti)
    cs_dst_t = _mm(jnp.sum(wgdst[...], axis=0, keepdims=True), ti)
    cs_v_t = _mm(jnp.sum(wgv[...], axis=0, keepdims=True), ti)
    gate = _sigmoid(cs_src_t + cs_dst_t + _mm(he256, bdge[...]))
    r_rep = _mm(rb8, t32)
    feats = jnp.exp(-_GAMMA_G * (r_rep - _cen256(0.0, 1.0)) ** 2)
    filt = _mm(feats, bdgbf[...])
    prod = gate * filt * cs_v_t * cut256
    agg = _mm(prod, summ)
    hn_out[...] = 1.0 + _silu(agg)


def _l01_body(rb_ref, rT_ref, cosT_ref, z2_ref, zj8_ref, zT_ref,
              zjT_ref, ee_ref, tr_ref,
              whe0, wsrcT0, bdd0, bdv0, bdbf0,
              whe1, wsrcT1, bdd1, bdv1, bdbf1,
              he1_out, he2_out):
    bn = rb_ref.shape[0]
    r2 = _K * bn
    z2 = z2_ref[...]
    zj8 = zj8_ref[...]
    ee = ee_ref[...]
    tr = tr_ref[...]
    eq_row, oh64, cut256, feats = _hshared(
        cosT_ref[...].reshape(r2, _K), z2, zj8,
        zT_ref[...].reshape(r2, 1), zjT_ref[...].reshape(r2, 1),
        rb_ref[...], rT_ref[...].reshape(r2, 1))
    ee0 = ee[0:1, :].astype(jnp.float32)
    ee1 = ee[1:2, :].astype(jnp.float32)
    he0_rows = eq_row * ee1 + (1.0 - eq_row) * ee0
    he0_256 = _he0_256(z2, zj8, ee)
    y0 = _hlayer(he0_rows, he0_256, oh64, cut256, feats, tr,
                 whe0[...], wsrcT0[...], bdd0[...], bdv0[...], bdbf0[...])
    he1_rows = he0_rows + y0
    he1_out[...] = he1_rows.reshape(_K, bn, _HID)
    he1_256 = _fold256(he1_rows, bn)
    y1 = _hlayer(he1_rows, he1_256, oh64, cut256, feats, tr,
                 whe1[...], wsrcT1[...], bdd1[...], bdv1[...], bdbf1[...])
    he2_out[...] = (he1_rows + y1).reshape(_K, bn, _HID)


def _heT_to_256(heT_ref):
    blk = heT_ref[...]
    return jnp.concatenate([blk[j] for j in range(_K)], axis=1)


def _glayer_body(rb_ref, heT_ref, hn_ref, xj_ref,
                 wsrcT, bdd, bde, bdv, bdbf, hn_out):
    bn = rb_ref.shape[0]
    hn_out[...] = _gconv256(rb_ref[...], _heT_to_256(heT_ref), hn_ref[...],
                            xj_ref[...], wsrcT[...], bdd[...], bde[...],
                            bdv[...], bdbf[...], bn)


def _gfinal_body(rb_ref, heT_ref, hn_ref, xj_ref,
                 wsrcT, bdd, bde, bdv, bdbf,
                 f1w, f1b, f2w, f2b, acc_out, *, n_total):
    i = pl.program_id(0)
    bn = rb_ref.shape[0]
    hn3 = _gconv256(rb_ref[...], _heT_to_256(heT_ref), hn_ref[...],
                    xj_ref[...],
                    wsrcT[...], bdd[...], bde[...], bdv[...], bdbf[...], bn)
    x = _silu(_mm(hn3, f1w[...]) + f1b[...])
    x2 = _silu(_mm(x, f2w[...]) + f2b[...])
    part = jnp.sum(x2, axis=0, keepdims=True) * (1.0 / n_total)

    @pl.when(i == 0)
    def _():
        acc_out[...] = jnp.zeros_like(acc_out)

    acc_out[...] += part


def kernel(r, cos, idx, z, edge_emb, trip_emb,
           Wg_src, Wg_dst, Wg_e, Wg_bf, Wg_v,
           Wh_src, Wh_dst, Wh_e, Wh_bf, Wh_v,
           fc1_w, fc1_b, fc2_w, fc2_b):
    n, k = r.shape
    bn = 200
    g = n // bn
    bng = 1000 if n % 1000 == 0 else bn
    gg = n // bng
    out_dim = fc2_w.shape[1]
    f32 = jnp.float32

    nflat = n * k
    ch = -(-nflat // (_NW * _CW))
    ipad = _NW * ch * _CW
    idx3 = jnp.pad(idx.reshape(-1), (0, ipad - nflat)).reshape(_NW, ch, _CW)

    z16 = jnp.broadcast_to(z[:, None], (n, 16))
    zj = _sc_gather(z16, idx3).reshape(ipad, 16)[:nflat, 0].reshape(n, k)

    z2 = z[:, None]
    cosT = jnp.transpose(cos, (1, 0, 2))
    rT3 = r.T[:, :, None]
    zT3 = jnp.broadcast_to(z[None, :, None], (k, n, 1))
    zjT3 = zj.T[:, :, None]

    eye8 = jnp.eye(8, dtype=f32)
    bd = lambda w: jnp.kron(eye8, w.astype(f32))
    tile8 = lambda w: jnp.tile(w.astype(f32), (1, 8))

    spec_rb = pl.BlockSpec((bn, k), lambda i: (i, 0))
    spec_rowT = pl.BlockSpec((k, bn, 1), lambda i: (0, i, 0))
    spec_cosT = pl.BlockSpec((k, bn, k), lambda i: (0, i, 0))
    spec_z2 = pl.BlockSpec((bn, 1), lambda i: (i, 0))
    spec_heT = pl.BlockSpec((k, bn, _HID), lambda i: (0, i, 0))

    def wspec(a):
        nd = a.ndim
        return pl.BlockSpec(a.shape, lambda i, _nd=nd: (0,) * _nd)

    spec_rbg = pl.BlockSpec((bng, k), lambda i: (i, 0))
    spec_z2g = pl.BlockSpec((bng, 1), lambda i: (i, 0))

    hn1 = pl.pallas_call(
        _l0g_body,
        grid=(gg,),
        in_specs=[spec_rbg, spec_z2g, spec_rbg, wspec(edge_emb),
                  pl.BlockSpec((_HID, _HID), lambda i: (0, 0)),
                  pl.BlockSpec((_HID, _HID), lambda i: (0, 0)),
                  pl.BlockSpec((_HID, _HID), lambda i: (0, 0)),
                  pl.BlockSpec((_KH, _KH), lambda i: (0, 0)),
                  pl.BlockSpec((_KH, _KH), lambda i: (0, 0))],
        out_specs=pl.BlockSpec((bng, _HID), lambda i: (i, 0)),
        out_shape=jax.ShapeDtypeStruct((n, _HID), f32),
    )(r, z2, zj, edge_emb,
      Wg_src[0], Wg_dst[0], Wg_v[0], bd(Wg_e[0]), bd(Wg_bf[0]))

    xj1 = _sc_gather(hn1, idx3).reshape(ipad // k, _KH)

    wsp_h = [pl.BlockSpec((_HID, _HID), lambda i: (0, 0)),
             pl.BlockSpec((_HID, _KH), lambda i: (0, 0)),
             pl.BlockSpec((_KH, _KH), lambda i: (0, 0)),
             pl.BlockSpec((_KH, _KH), lambda i: (0, 0)),
             pl.BlockSpec((_KH, _KH), lambda i: (0, 0))]
    he1, he2 = pl.pallas_call(
        _l01_body,
        grid=(g,),
        in_specs=[spec_rb, spec_rowT, spec_cosT, spec_z2, spec_rb,
                  spec_rowT, spec_rowT, wspec(edge_emb), wspec(trip_emb)]
                 + wsp_h + wsp_h,
        out_specs=[spec_heT, spec_heT],
        out_shape=[jax.ShapeDtypeStruct((k, n, _HID), f32),
                   jax.ShapeDtypeStruct((k, n, _HID), f32)],
    )(r, rT3, cosT, z2, zj, zT3, zjT3, edge_emb, trip_emb,
      Wh_e[0], tile8(Wh_src[0]), bd(Wh_dst[0]), bd(Wh_v[0]), bd(Wh_bf[0]),
      Wh_e[1], tile8(Wh_src[1]), bd(Wh_dst[1]), bd(Wh_v[1]), bd(Wh_bf[1]))
    spec_he256g = pl.BlockSpec((bng, _KH), lambda i: (i, 0))
    spec_hng = pl.BlockSpec((bng, _HID), lambda i: (i, 0))
    spec_heTg = pl.BlockSpec((k, bng, _HID), lambda i: (0, i, 0))

    hn2 = pl.pallas_call(
        _glayer_body,
        grid=(gg,),
        in_specs=[spec_rbg, spec_heTg, spec_hng, spec_he256g,
                  pl.BlockSpec((_HID, _KH), lambda i: (0, 0)),
                  pl.BlockSpec((_KH, _KH), lambda i: (0, 0)),
                  pl.BlockSpec((_KH, _KH), lambda i: (0, 0)),
                  pl.BlockSpec((_KH, _KH), lambda i: (0, 0)),
                  pl.BlockSpec((_KH, _KH), lambda i: (0, 0))],
        out_specs=spec_hng,
        out_shape=jax.ShapeDtypeStruct((n, _HID), f32),
    )(r, he1, hn1, xj1,
      tile8(Wg_src[1]), bd(Wg_dst[1]), bd(Wg_e[1]), bd(Wg_v[1]),
      bd(Wg_bf[1]))

    xj2 = _sc_gather(hn2, idx3).reshape(ipad // k, _KH)

    acc = pl.pallas_call(
        functools.partial(_gfinal_body, n_total=float(n)),
        grid=(gg,),
        in_specs=[spec_rbg, spec_heTg, spec_hng, spec_he256g,
                  pl.BlockSpec((_HID, _KH), lambda i: (0, 0)),
                  pl.BlockSpec((_KH, _KH), lambda i: (0, 0)),
                  pl.BlockSpec((_KH, _KH), lambda i: (0, 0)),
                  pl.BlockSpec((_KH, _KH), lambda i: (0, 0)),
                  pl.BlockSpec((_KH, _KH), lambda i: (0, 0)),
                  wspec(fc1_w), pl.BlockSpec((1, _HID), lambda i: (0, 0)),
                  wspec(fc2_w), pl.BlockSpec((1, out_dim), lambda i: (0, 0))],
        out_specs=pl.BlockSpec((1, out_dim), lambda i: (0, 0)),
        out_shape=jax.ShapeDtypeStruct((1, out_dim), f32),
    )(r, he2, hn2, xj2,
      tile8(Wg_src[2]), bd(Wg_dst[2]), bd(Wg_e[2]), bd(Wg_v[2]),
      bd(Wg_bf[2]),
      fc1_w, fc1_b[None, :], fc2_w, fc2_b[None, :])

    return acc[0]


# speedup vs baseline: 4.7771x; 1.0231x over previous
import functools

import jax
import jax.numpy as jnp
from jax import lax
from jax.experimental import pallas as pl
from jax.experimental.pallas import tpu as pltpu
from jax.experimental.pallas import tpu_sc as plsc

_K = 8
_HID = 32
_RAD = 32
_KH = _K * _HID
_NC = 2
_NS = 16
_NW = _NC * _NS
_CW = 128

_GAMMA_G = float((_RAD - 1) ** 2)
_GAMMA_H = float(((_RAD - 1) / 2.0) ** 2)


def _sc_gather(table, idx3):
    nw, ch, cw = idx3.shape
    d = table.shape[1]
    mesh = plsc.VectorSubcoreMesh(core_axis_name="c", subcore_axis_name="s")

    @functools.partial(
        pl.kernel,
        out_type=jax.ShapeDtypeStruct((nw, ch, cw, d), table.dtype),
        mesh=mesh,
        compiler_params=pltpu.CompilerParams(use_tc_tiling_on_sc=False),
        scratch_types=[
            pltpu.VMEM((ch, cw), jnp.int32),
            pltpu.VMEM((ch, cw, d), table.dtype),
            pltpu.SemaphoreType.DMA,
        ],
    )
    def gk(table_hbm, idx_hbm, out_hbm, idx_v, rows_v, sem):
        wid = lax.axis_index("s") * _NC + lax.axis_index("c")
        pltpu.sync_copy(idx_hbm.at[wid], idx_v)
        copies = [
            pltpu.async_copy(table_hbm.at[idx_v.at[j]], rows_v.at[j], sem)
            for j in range(ch)
        ]
        for j, c in enumerate(copies):
            c.wait()
            pltpu.sync_copy(rows_v.at[j], out_hbm.at[wid, j])

    return gk(table, idx3)


def _mm(a, w):
    return jnp.dot(a.astype(jnp.bfloat16), w.astype(jnp.bfloat16),
                   preferred_element_type=jnp.float32)


def _sigmoid(x):
    return 1.0 / (1.0 + jnp.exp(-x))


def _silu(x):
    return x * _sigmoid(x)


def _cutoff(rr):
    u = jnp.clip((rr - 0.95) / 0.05, 0.0, 1.0)
    return 0.5 * (jnp.cos(jnp.pi * u) + 1.0)


def _iota2(shape, dim):
    return lax.broadcasted_iota(jnp.int32, shape, dim)


def _vtile8(x):
    m, l = x.shape
    return jnp.broadcast_to(x[None], (8, m, l)).reshape(8 * m, l)


def _fold256(x_rows, m):
    return jnp.concatenate([x_rows[j * m:(j + 1) * m, :] for j in range(8)],
                           axis=1)


def _consts():
    t32 = (_iota2((_K, _KH), 1) // _HID == _iota2((_K, _KH), 0)
           ).astype(jnp.float32)
    t8 = (_iota2((_K, 64), 1) // 8 == _iota2((_K, 64), 0)).astype(jnp.float32)
    a64 = (_iota2((64, 8), 0) % 8 == _iota2((64, 8), 1)).astype(jnp.float32)
    ti = (_iota2((_HID, _KH), 1) % _HID == _iota2((_HID, _KH), 0)
          ).astype(jnp.float32)
    summ = (_iota2((_KH, _HID), 0) % _HID == _iota2((_KH, _HID), 1)
            ).astype(jnp.float32)
    mask64 = (_iota2((64, _KH), 0) // 8 == _iota2((64, _KH), 1) // _HID
              ).astype(jnp.float32)
    return t32, t8, a64, ti, summ, mask64


def _cen256(lo, hi):
    step = (hi - lo) / (_RAD - 1)
    return lo + step * (_iota2((1, _KH), 1) % _RAD).astype(jnp.float32)


def _hshared(cos_rows, z2, zj8, zrowT, zjrowT, rb8, r_rowT):
    t32, t8, _, _, _, _ = _consts()
    cg8 = _cutoff(rb8)
    cg_lane = _vtile8(_mm(cg8, t32))
    cut256 = jnp.minimum(cg_lane, _cutoff(r_rowT))
    eq_row = (zrowT == zjrowT).astype(jnp.float32)
    eq_lane = _vtile8((z2 == zj8).astype(jnp.float32))
    eq_jk = (zjrowT == _vtile8(zj8)).astype(jnp.float32)
    t8v = eq_row * 4.0 + eq_lane * 2.0 + eq_jk
    t_rep = _mm(t8v, t8)
    c64 = (_iota2((1, 64), 1) % 8).astype(jnp.float32)
    oh64 = (t_rep == c64).astype(jnp.float32)
    cos_rep = _mm(cos_rows, t32)
    feats = jnp.exp(-_GAMMA_H * (cos_rep - _cen256(-1.0, 1.0)) ** 2)
    return eq_row, oh64, cut256, feats


def _hlayer(he_rows, he256, oh64, cut256, feats,
            trip, whe, wsrcT, bdd, bdv, bdbf):
    _, _, a64, ti, summ, mask64 = _consts()
    te_tab = _mm(trip, whe)
    te_bd = _mm(_mm(a64, te_tab), ti) * mask64
    te256 = _mm(oh64, te_bd)
    ms_t = _mm(he_rows, wsrcT)
    md256 = _vtile8(_mm(he256, bdd))
    vh256 = _vtile8(_mm(he256, bdv))
    filt = _mm(feats, bdbf)
    gate = _sigmoid(ms_t + md256 + te256)
    prod = gate * filt * vh256 * cut256
    agg = _mm(prod, summ)
    return _silu(agg)


def _gconv256(rb8, he256, hn, xj256, wsrcT, bdd, bde, bdv, bdbf, bn):
    t32, _, _, _, summ, _ = _consts()
    cg8 = _cutoff(rb8)
    cut256 = _mm(cg8, t32)
    sg_t = _mm(hn, wsrcT)
    gate = _sigmoid(sg_t + _mm(xj256, bdd) + _mm(he256, bde))
    r_rep = _mm(rb8, t32)
    feats = jnp.exp(-_GAMMA_G * (r_rep - _cen256(0.0, 1.0)) ** 2)
    filt = _mm(feats, bdbf)
    vg = _mm(xj256, bdv)
    prod = gate * filt * vg * cut256
    agg = _mm(prod, summ)
    return hn + _silu(agg)


def _he0_256(z2, zj8, ee):
    t32, _, _, ti, _, _ = _consts()
    eq256 = _mm((z2 == zj8).astype(jnp.float32), t32)
    return eq256 * _mm(ee[1:2, :], ti) + (1.0 - eq256) * _mm(ee[0:1, :], ti)


def _l0g_body(rb_ref, z2_ref, zj8_ref, ee_ref, wgsrc, wgdst, wgv, bdge, bdgbf,
              hn_out):
    t32, _, _, ti, summ, _ = _consts()
    rb8 = rb_ref[...]
    he256 = _he0_256(z2_ref[...], zj8_ref[...], ee_ref[...])
    cg8 = _cutoff(rb8)
    cut256 = _mm(cg8, t32)
    cs_src_t = _mm(jnp.sum(wgsrc[...], axis=0, keepdims=True), ti)
    cs_dst_t = _mm(jnp.sum(wgdst[...], axis=0, keepdims=True), ti)
    cs_v_t = _mm(jnp.sum(wgv[...], axis=0, keepdims=True), ti)
    gate = _sigmoid(cs_src_t + cs_dst_t + _mm(he256, bdge[...]))
    r_rep = _mm(rb8, t32)
    feats = jnp.exp(-_GAMMA_G * (r_rep - _cen256(0.0, 1.0)) ** 2)
    filt = _mm(feats, bdgbf[...])
    prod = gate * filt * cs_v_t * cut256
    agg = _mm(prod, summ)
    hn_out[...] = 1.0 + _silu(agg)


def _l01_body(rb_ref, rT_ref, cosT_ref, z2_ref, zj8_ref, zT_ref,
              zjT_ref, ee_ref, tr_ref,
              whe0, wsrcT0, bdd0, bdv0, bdbf0,
              whe1, wsrcT1, bdd1, bdv1, bdbf1,
              he1_out, he2_out):
    bn = rb_ref.shape[0]
    r2 = _K * bn
    z2 = z2_ref[...]
    zj8 = zj8_ref[...]
    ee = ee_ref[...]
    tr = tr_ref[...]
    eq_row, oh64, cut256, feats = _hshared(
        cosT_ref[...].reshape(r2, _K), z2, zj8,
        zT_ref[...].reshape(r2, 1), zjT_ref[...].reshape(r2, 1),
        rb_ref[...], rT_ref[...].reshape(r2, 1))
    ee0 = ee[0:1, :].astype(jnp.float32)
    ee1 = ee[1:2, :].astype(jnp.float32)
    he0_rows = eq_row * ee1 + (1.0 - eq_row) * ee0
    he0_256 = _he0_256(z2, zj8, ee)
    y0 = _hlayer(he0_rows, he0_256, oh64, cut256, feats, tr,
                 whe0[...], wsrcT0[...], bdd0[...], bdv0[...], bdbf0[...])
    he1_rows = he0_rows + y0
    he1_out[...] = he1_rows.reshape(_K, bn, _HID)
    he1_256 = _fold256(he1_rows, bn)
    y1 = _hlayer(he1_rows, he1_256, oh64, cut256, feats, tr,
                 whe1[...], wsrcT1[...], bdd1[...], bdv1[...], bdbf1[...])
    he2_out[...] = (he1_rows + y1).reshape(_K, bn, _HID)


def _heT_to_256(heT_ref):
    blk = heT_ref[...]
    return jnp.concatenate([blk[j] for j in range(_K)], axis=1)


def _glayer_body(rb_ref, heT_ref, hn_ref, xj_ref,
                 wsrcT, bdd, bde, bdv, bdbf, hn_out):
    bn = rb_ref.shape[0]
    hn_out[...] = _gconv256(rb_ref[...], _heT_to_256(heT_ref), hn_ref[...],
                            xj_ref[...], wsrcT[...], bdd[...], bde[...],
                            bdv[...], bdbf[...], bn)


def _gfinal_body(rb_ref, heT_ref, hn_ref, xj_ref,
                 wsrcT, bdd, bde, bdv, bdbf,
                 f1w, f1b, f2w, f2b, acc_out, *, n_total):
    i = pl.program_id(0)
    bn = rb_ref.shape[0]
    hn3 = _gconv256(rb_ref[...], _heT_to_256(heT_ref), hn_ref[...],
                    xj_ref[...],
                    wsrcT[...], bdd[...], bde[...], bdv[...], bdbf[...], bn)
    x = _silu(_mm(hn3, f1w[...]) + f1b[...])
    x2 = _silu(_mm(x, f2w[...]) + f2b[...])
    part = jnp.sum(x2, axis=0, keepdims=True) * (1.0 / n_total)

    @pl.when(i == 0)
    def _():
        acc_out[...] = jnp.zeros_like(acc_out)

    acc_out[...] += part


def kernel(r, cos, idx, z, edge_emb, trip_emb,
           Wg_src, Wg_dst, Wg_e, Wg_bf, Wg_v,
           Wh_src, Wh_dst, Wh_e, Wh_bf, Wh_v,
           fc1_w, fc1_b, fc2_w, fc2_b):
    n, k = r.shape
    bn = 400 if n % 400 == 0 else 200
    g = n // bn
    bng = 1000 if n % 1000 == 0 else bn
    gg = n // bng
    out_dim = fc2_w.shape[1]
    f32 = jnp.float32

    nflat = n * k
    ch = -(-nflat // (_NW * _CW))
    ipad = _NW * ch * _CW
    idx3 = jnp.pad(idx.reshape(-1), (0, ipad - nflat)).reshape(_NW, ch, _CW)

    z16 = jnp.broadcast_to(z[:, None], (n, 16))
    zj = _sc_gather(z16, idx3).reshape(ipad, 16)[:nflat, 0].reshape(n, k)

    z2 = z[:, None]
    cosT = jnp.transpose(cos, (1, 0, 2))
    rT3 = r.T[:, :, None]
    zT3 = jnp.broadcast_to(z[None, :, None], (k, n, 1))
    zjT3 = zj.T[:, :, None]

    eye8 = jnp.eye(8, dtype=f32)
    bd = lambda w: jnp.kron(eye8, w.astype(f32))
    tile8 = lambda w: jnp.tile(w.astype(f32), (1, 8))

    spec_rb = pl.BlockSpec((bn, k), lambda i: (i, 0))
    spec_rowT = pl.BlockSpec((k, bn, 1), lambda i: (0, i, 0))
    spec_cosT = pl.BlockSpec((k, bn, k), lambda i: (0, i, 0))
    spec_z2 = pl.BlockSpec((bn, 1), lambda i: (i, 0))
    spec_heT = pl.BlockSpec((k, bn, _HID), lambda i: (0, i, 0))

    def wspec(a):
        nd = a.ndim
        return pl.BlockSpec(a.shape, lambda i, _nd=nd: (0,) * _nd)

    spec_rbg = pl.BlockSpec((bng, k), lambda i: (i, 0))
    spec_z2g = pl.BlockSpec((bng, 1), lambda i: (i, 0))

    hn1 = pl.pallas_call(
        _l0g_body,
        grid=(gg,),
        in_specs=[spec_rbg, spec_z2g, spec_rbg, wspec(edge_emb),
                  pl.BlockSpec((_HID, _HID), lambda i: (0, 0)),
                  pl.BlockSpec((_HID, _HID), lambda i: (0, 0)),
                  pl.BlockSpec((_HID, _HID), lambda i: (0, 0)),
                  pl.BlockSpec((_KH, _KH), lambda i: (0, 0)),
                  pl.BlockSpec((_KH, _KH), lambda i: (0, 0))],
        out_specs=pl.BlockSpec((bng, _HID), lambda i: (i, 0)),
        out_shape=jax.ShapeDtypeStruct((n, _HID), f32),
    )(r, z2, zj, edge_emb,
      Wg_src[0], Wg_dst[0], Wg_v[0], bd(Wg_e[0]), bd(Wg_bf[0]))

    xj1 = _sc_gather(hn1, idx3).reshape(ipad // k, _KH)

    wsp_h = [pl.BlockSpec((_HID, _HID), lambda i: (0, 0)),
             pl.BlockSpec((_HID, _KH), lambda i: (0, 0)),
             pl.BlockSpec((_KH, _KH), lambda i: (0, 0)),
             pl.BlockSpec((_KH, _KH), lambda i: (0, 0)),
             pl.BlockSpec((_KH, _KH), lambda i: (0, 0))]
    he1, he2 = pl.pallas_call(
        _l01_body,
        grid=(g,),
        in_specs=[spec_rb, spec_rowT, spec_cosT, spec_z2, spec_rb,
                  spec_rowT, spec_rowT, wspec(edge_emb), wspec(trip_emb)]
                 + wsp_h + wsp_h,
        out_specs=[spec_heT, spec_heT],
        out_shape=[jax.ShapeDtypeStruct((k, n, _HID), f32),
                   jax.ShapeDtypeStruct((k, n, _HID), f32)],
    )(r, rT3, cosT, z2, zj, zT3, zjT3, edge_emb, trip_emb,
      Wh_e[0], tile8(Wh_src[0]), bd(Wh_dst[0]), bd(Wh_v[0]), bd(Wh_bf[0]),
      Wh_e[1], tile8(Wh_src[1]), bd(Wh_dst[1]), bd(Wh_v[1]), bd(Wh_bf[1]))
    spec_he256g = pl.BlockSpec((bng, _KH), lambda i: (i, 0))
    spec_hng = pl.BlockSpec((bng, _HID), lambda i: (i, 0))
    spec_heTg = pl.BlockSpec((k, bng, _HID), lambda i: (0, i, 0))

    hn2 = pl.pallas_call(
        _glayer_body,
        grid=(gg,),
        in_specs=[spec_rbg, spec_heTg, spec_hng, spec_he256g,
                  pl.BlockSpec((_HID, _KH), lambda i: (0, 0)),
                  pl.BlockSpec((_KH, _KH), lambda i: (0, 0)),
                  pl.BlockSpec((_KH, _KH), lambda i: (0, 0)),
                  pl.BlockSpec((_KH, _KH), lambda i: (0, 0)),
                  pl.BlockSpec((_KH, _KH), lambda i: (0, 0))],
        out_specs=spec_hng,
        out_shape=jax.ShapeDtypeStruct((n, _HID), f32),
    )(r, he1, hn1, xj1,
      tile8(Wg_src[1]), bd(Wg_dst[1]), bd(Wg_e[1]), bd(Wg_v[1]),
      bd(Wg_bf[1]))

    xj2 = _sc_gather(hn2, idx3).reshape(ipad // k, _KH)

    acc = pl.pallas_call(
        functools.partial(_gfinal_body, n_total=float(n)),
        grid=(gg,),
        in_specs=[spec_rbg, spec_heTg, spec_hng, spec_he256g,
                  pl.BlockSpec((_HID, _KH), lambda i: (0, 0)),
                  pl.BlockSpec((_KH, _KH), lambda i: (0, 0)),
                  pl.BlockSpec((_KH, _KH), lambda i: (0, 0)),
                  pl.BlockSpec((_KH, _KH), lambda i: (0, 0)),
                  pl.BlockSpec((_KH, _KH), lambda i: (0, 0)),
                  wspec(fc1_w), pl.BlockSpec((1, _HID), lambda i: (0, 0)),
                  wspec(fc2_w), pl.BlockSpec((1, out_dim), lambda i: (0, 0))],
        out_specs=pl.BlockSpec((1, out_dim), lambda i: (0, 0)),
        out_shape=jax.ShapeDtypeStruct((1, out_dim), f32),
    )(r, he2, hn2, xj2,
      tile8(Wg_src[2]), bd(Wg_dst[2]), bd(Wg_e[2]), bd(Wg_v[2]),
      bd(Wg_bf[2]),
      fc1_w, fc1_b[None, :], fc2_w, fc2_b[None, :])

    return acc[0]


# speedup vs baseline: 6.7097x; 1.4046x over previous
import functools

import jax
import jax.numpy as jnp
from jax import lax
from jax.experimental import pallas as pl
from jax.experimental.pallas import tpu as pltpu
from jax.experimental.pallas import tpu_sc as plsc

_K = 8
_HID = 32
_RAD = 32
_KH = _K * _HID
_NC = 2
_NS = 16
_NW = _NC * _NS
_CW = 128

_GAMMA_G = float((_RAD - 1) ** 2)
_GAMMA_H = float(((_RAD - 1) / 2.0) ** 2)


def _sc_gather(table, idx3):
    nw, ch, cw = idx3.shape
    d = table.shape[1]
    mesh = plsc.VectorSubcoreMesh(core_axis_name="c", subcore_axis_name="s")

    @functools.partial(
        pl.kernel,
        out_type=jax.ShapeDtypeStruct((nw, ch, cw, d), table.dtype),
        mesh=mesh,
        compiler_params=pltpu.CompilerParams(use_tc_tiling_on_sc=False),
        scratch_types=[
            pltpu.VMEM((ch, cw), jnp.int32),
            pltpu.VMEM((ch, cw, d), table.dtype),
            pltpu.SemaphoreType.DMA,
        ],
    )
    def gk(table_hbm, idx_hbm, out_hbm, idx_v, rows_v, sem):
        wid = lax.axis_index("s") * _NC + lax.axis_index("c")
        pltpu.sync_copy(idx_hbm.at[wid], idx_v)
        copies = [
            pltpu.async_copy(table_hbm.at[idx_v.at[j]], rows_v.at[j], sem)
            for j in range(ch)
        ]
        for j, c in enumerate(copies):
            c.wait()
            pltpu.sync_copy(rows_v.at[j], out_hbm.at[wid, j])

    return gk(table, idx3)


def _mm(a, w):
    return jnp.dot(a.astype(jnp.bfloat16), w.astype(jnp.bfloat16),
                   preferred_element_type=jnp.float32)


def _sigmoid(x):
    return 1.0 / (1.0 + jnp.exp(-x))


def _silu(x):
    return x * _sigmoid(x)


def _cutoff(rr):
    u = jnp.clip((rr - 0.95) / 0.05, 0.0, 1.0)
    return 0.5 * (jnp.cos(jnp.pi * u) + 1.0)


def _iota2(shape, dim):
    return lax.broadcasted_iota(jnp.int32, shape, dim)


def _vtile8(x):
    m, l = x.shape
    return jnp.broadcast_to(x[None], (8, m, l)).reshape(8 * m, l)


def _fold256(x_rows, m):
    return jnp.concatenate([x_rows[j * m:(j + 1) * m, :] for j in range(8)],
                           axis=1)


def _consts():
    t32 = (_iota2((_K, _KH), 1) // _HID == _iota2((_K, _KH), 0)
           ).astype(jnp.float32)
    t8 = (_iota2((_K, 64), 1) // 8 == _iota2((_K, 64), 0)).astype(jnp.float32)
    a64 = (_iota2((64, 8), 0) % 8 == _iota2((64, 8), 1)).astype(jnp.float32)
    ti = (_iota2((_HID, _KH), 1) % _HID == _iota2((_HID, _KH), 0)
          ).astype(jnp.float32)
    summ = (_iota2((_KH, _HID), 0) % _HID == _iota2((_KH, _HID), 1)
            ).astype(jnp.float32)
    mask64 = (_iota2((64, _KH), 0) // 8 == _iota2((64, _KH), 1) // _HID
              ).astype(jnp.float32)
    return t32, t8, a64, ti, summ, mask64


def _cen256(lo, hi):
    step = (hi - lo) / (_RAD - 1)
    return lo + step * (_iota2((1, _KH), 1) % _RAD).astype(jnp.float32)


def _rowsel(x8, width):
    pieces = []
    for i in range(8):
        si = (_iota2((_K, width), 0) == i).astype(jnp.float32)
        pieces.append(_mm(x8, si))
    return jnp.concatenate(pieces, axis=0)


def _hshared(cos_rows, z2, zj8, rb8):
    t32, t8, _, _, _, _ = _consts()
    cg8 = _cutoff(rb8)
    cut256 = jnp.minimum(_vtile8(_mm(cg8, t32)), _rowsel(cg8, _KH))
    eq8 = (z2 == zj8).astype(jnp.float32)
    zj8f = zj8.astype(jnp.float32)
    eq_row32 = _rowsel(eq8, _HID)
    eq_row8 = _rowsel(eq8, _K)
    eq_lane = _vtile8(eq8)
    eq_jk = (_rowsel(zj8f, _K) == _vtile8(zj8f)).astype(jnp.float32)
    t8v = eq_row8 * 4.0 + eq_lane * 2.0 + eq_jk
    t_rep = _mm(t8v, t8)
    c64 = (_iota2((1, 64), 1) % 8).astype(jnp.float32)
    oh64 = (t_rep == c64).astype(jnp.float32)
    cos_rep = _mm(cos_rows, t32)
    feats = jnp.exp(-_GAMMA_H * (cos_rep - _cen256(-1.0, 1.0)) ** 2)
    return eq_row32, oh64, cut256, feats


def _hlayer(he_rows, he256, oh64, cut256, feats,
            trip, whe, wsrcT, bdd, bdv, bdbf):
    _, _, a64, ti, summ, mask64 = _consts()
    te_tab = _mm(trip, whe)
    te_bd = _mm(_mm(a64, te_tab), ti) * mask64
    te256 = _mm(oh64, te_bd)
    ms_t = _mm(he_rows, wsrcT)
    md256 = _vtile8(_mm(he256, bdd))
    vh256 = _vtile8(_mm(he256, bdv))
    filt = _mm(feats, bdbf)
    gate = _sigmoid(ms_t + md256 + te256)
    prod = gate * filt * vh256 * cut256
    agg = _mm(prod, summ)
    return _silu(agg)


def _gconv256(rb8, he256, hn, xj256, wsrcT, bdd, bde, bdv, bdbf, bn):
    t32, _, _, _, summ, _ = _consts()
    cg8 = _cutoff(rb8)
    cut256 = _mm(cg8, t32)
    sg_t = _mm(hn, wsrcT)
    gate = _sigmoid(sg_t + _mm(xj256, bdd) + _mm(he256, bde))
    r_rep = _mm(rb8, t32)
    feats = jnp.exp(-_GAMMA_G * (r_rep - _cen256(0.0, 1.0)) ** 2)
    filt = _mm(feats, bdbf)
    vg = _mm(xj256, bdv)
    prod = gate * filt * vg * cut256
    agg = _mm(prod, summ)
    return hn + _silu(agg)


def _he0_256(z2, zj8, ee):
    t32, _, _, ti, _, _ = _consts()
    eq256 = _mm((z2 == zj8).astype(jnp.float32), t32)
    return eq256 * _mm(ee[1:2, :], ti) + (1.0 - eq256) * _mm(ee[0:1, :], ti)


def _l0g_body(rb_ref, z2_ref, zj8_ref, ee_ref, wgsrc, wgdst, wgv, bdge, bdgbf,
              hn_out):
    t32, _, _, ti, summ, _ = _consts()
    rb8 = rb_ref[...]
    he256 = _he0_256(z2_ref[...], zj8_ref[...], ee_ref[...])
    cg8 = _cutoff(rb8)
    cut256 = _mm(cg8, t32)
    cs_src_t = _mm(jnp.sum(wgsrc[...], axis=0, keepdims=True), ti)
    cs_dst_t = _mm(jnp.sum(wgdst[...], axis=0, keepdims=True), ti)
    cs_v_t = _mm(jnp.sum(wgv[...], axis=0, keepdims=True), ti)
    gate = _sigmoid(cs_src_t + cs_dst_t + _mm(he256, bdge[...]))
    r_rep = _mm(rb8, t32)
    feats = jnp.exp(-_GAMMA_G * (r_rep - _cen256(0.0, 1.0)) ** 2)
    filt = _mm(feats, bdgbf[...])
    prod = gate * filt * cs_v_t * cut256
    agg = _mm(prod, summ)
    hn_out[...] = 1.0 + _silu(agg)


def _l01_body(rb_ref, cosT_ref, z2_ref, zj8_ref, ee_ref, tr_ref,
              whe0, wsrcT0, bdd0, bdv0, bdbf0,
              whe1, wsrcT1, bdd1, bdv1, bdbf1,
              he1_out, he2_out):
    bn = rb_ref.shape[0]
    r2 = _K * bn
    z2 = z2_ref[...]
    zj8 = zj8_ref[...]
    ee = ee_ref[...]
    tr = tr_ref[...]
    eq_row32, oh64, cut256, feats = _hshared(
        cosT_ref[...].reshape(r2, _K), z2, zj8, rb_ref[...])
    ee0 = ee[0:1, :].astype(jnp.float32)
    ee1 = ee[1:2, :].astype(jnp.float32)
    he0_rows = eq_row32 * (ee1 - ee0) + ee0
    he0_256 = _he0_256(z2, zj8, ee)
    y0 = _hlayer(he0_rows, he0_256, oh64, cut256, feats, tr,
                 whe0[...], wsrcT0[...], bdd0[...], bdv0[...], bdbf0[...])
    he1_rows = he0_rows + y0
    he1_out[...] = he1_rows.reshape(_K, bn, _HID)
    he1_256 = _fold256(he1_rows, bn)
    y1 = _hlayer(he1_rows, he1_256, oh64, cut256, feats, tr,
                 whe1[...], wsrcT1[...], bdd1[...], bdv1[...], bdbf1[...])
    he2_out[...] = (he1_rows + y1).reshape(_K, bn, _HID)


def _heT_to_256(heT_ref):
    blk = heT_ref[...]
    return jnp.concatenate([blk[j] for j in range(_K)], axis=1)


def _glayer_body(rb_ref, heT_ref, hn_ref, xj_ref,
                 wsrcT, bdd, bde, bdv, bdbf, hn_out):
    bn = rb_ref.shape[0]
    hn_out[...] = _gconv256(rb_ref[...], _heT_to_256(heT_ref), hn_ref[...],
                            xj_ref[...], wsrcT[...], bdd[...], bde[...],
                            bdv[...], bdbf[...], bn)


def _gfinal_body(rb_ref, heT_ref, hn_ref, xj_ref,
                 wsrcT, bdd, bde, bdv, bdbf,
                 f1w, f1b, f2w, f2b, acc_out, *, n_total):
    i = pl.program_id(0)
    bn = rb_ref.shape[0]
    hn3 = _gconv256(rb_ref[...], _heT_to_256(heT_ref), hn_ref[...],
                    xj_ref[...],
                    wsrcT[...], bdd[...], bde[...], bdv[...], bdbf[...], bn)
    x = _silu(_mm(hn3, f1w[...]) + f1b[...])
    x2 = _silu(_mm(x, f2w[...]) + f2b[...])
    part = jnp.sum(x2, axis=0, keepdims=True) * (1.0 / n_total)

    @pl.when(i == 0)
    def _():
        acc_out[...] = jnp.zeros_like(acc_out)

    acc_out[...] += part


def kernel(r, cos, idx, z, edge_emb, trip_emb,
           Wg_src, Wg_dst, Wg_e, Wg_bf, Wg_v,
           Wh_src, Wh_dst, Wh_e, Wh_bf, Wh_v,
           fc1_w, fc1_b, fc2_w, fc2_b):
    n, k = r.shape
    bn = 400 if n % 400 == 0 else 200
    g = n // bn
    bng = 1000 if n % 1000 == 0 else bn
    gg = n // bng
    out_dim = fc2_w.shape[1]
    f32 = jnp.float32

    nflat = n * k
    ch = -(-nflat // (_NW * _CW))
    ipad = _NW * ch * _CW
    idx3 = jnp.pad(idx.reshape(-1), (0, ipad - nflat)).reshape(_NW, ch, _CW)

    z16 = jnp.broadcast_to(z[:, None], (n, 16))
    zj = _sc_gather(z16, idx3).reshape(ipad, 16)[:nflat, 0].reshape(n, k)

    z2 = z[:, None]
    cosT = jnp.transpose(cos, (1, 0, 2))

    eye8 = jnp.eye(8, dtype=f32)
    bd = lambda w: jnp.kron(eye8, w.astype(f32))
    tile8 = lambda w: jnp.tile(w.astype(f32), (1, 8))

    spec_rb = pl.BlockSpec((bn, k), lambda i: (i, 0))
    spec_rowT = pl.BlockSpec((k, bn, 1), lambda i: (0, i, 0))
    spec_cosT = pl.BlockSpec((k, bn, k), lambda i: (0, i, 0))
    spec_z2 = pl.BlockSpec((bn, 1), lambda i: (i, 0))
    spec_heT = pl.BlockSpec((k, bn, _HID), lambda i: (0, i, 0))

    def wspec(a):
        nd = a.ndim
        return pl.BlockSpec(a.shape, lambda i, _nd=nd: (0,) * _nd)

    spec_rbg = pl.BlockSpec((bng, k), lambda i: (i, 0))
    spec_z2g = pl.BlockSpec((bng, 1), lambda i: (i, 0))

    hn1 = pl.pallas_call(
        _l0g_body,
        grid=(gg,),
        in_specs=[spec_rbg, spec_z2g, spec_rbg, wspec(edge_emb),
                  pl.BlockSpec((_HID, _HID), lambda i: (0, 0)),
                  pl.BlockSpec((_HID, _HID), lambda i: (0, 0)),
                  pl.BlockSpec((_HID, _HID), lambda i: (0, 0)),
                  pl.BlockSpec((_KH, _KH), lambda i: (0, 0)),
                  pl.BlockSpec((_KH, _KH), lambda i: (0, 0))],
        out_specs=pl.BlockSpec((bng, _HID), lambda i: (i, 0)),
        out_shape=jax.ShapeDtypeStruct((n, _HID), f32),
    )(r, z2, zj, edge_emb,
      Wg_src[0], Wg_dst[0], Wg_v[0], bd(Wg_e[0]), bd(Wg_bf[0]))

    xj1 = _sc_gather(hn1, idx3).reshape(ipad // k, _KH)

    wsp_h = [pl.BlockSpec((_HID, _HID), lambda i: (0, 0)),
             pl.BlockSpec((_HID, _KH), lambda i: (0, 0)),
             pl.BlockSpec((_KH, _KH), lambda i: (0, 0)),
             pl.BlockSpec((_KH, _KH), lambda i: (0, 0)),
             pl.BlockSpec((_KH, _KH), lambda i: (0, 0))]
    he1, he2 = pl.pallas_call(
        _l01_body,
        grid=(g,),
        in_specs=[spec_rb, spec_cosT, spec_z2, spec_rb,
                  wspec(edge_emb), wspec(trip_emb)]
                 + wsp_h + wsp_h,
        out_specs=[spec_heT, spec_heT],
        out_shape=[jax.ShapeDtypeStruct((k, n, _HID), f32),
                   jax.ShapeDtypeStruct((k, n, _HID), f32)],
    )(r, cosT, z2, zj, edge_emb, trip_emb,
      Wh_e[0], tile8(Wh_src[0]), bd(Wh_dst[0]), bd(Wh_v[0]), bd(Wh_bf[0]),
      Wh_e[1], tile8(Wh_src[1]), bd(Wh_dst[1]), bd(Wh_v[1]), bd(Wh_bf[1]))
    spec_he256g = pl.BlockSpec((bng, _KH), lambda i: (i, 0))
    spec_hng = pl.BlockSpec((bng, _HID), lambda i: (i, 0))
    spec_heTg = pl.BlockSpec((k, bng, _HID), lambda i: (0, i, 0))

    hn2 = pl.pallas_call(
        _glayer_body,
        grid=(gg,),
        in_specs=[spec_rbg, spec_heTg, spec_hng, spec_he256g,
                  pl.BlockSpec((_HID, _KH), lambda i: (0, 0)),
                  pl.BlockSpec((_KH, _KH), lambda i: (0, 0)),
                  pl.BlockSpec((_KH, _KH), lambda i: (0, 0)),
                  pl.BlockSpec((_KH, _KH), lambda i: (0, 0)),
                  pl.BlockSpec((_KH, _KH), lambda i: (0, 0))],
        out_specs=spec_hng,
        out_shape=jax.ShapeDtypeStruct((n, _HID), f32),
    )(r, he1, hn1, xj1,
      tile8(Wg_src[1]), bd(Wg_dst[1]), bd(Wg_e[1]), bd(Wg_v[1]),
      bd(Wg_bf[1]))

    xj2 = _sc_gather(hn2, idx3).reshape(ipad // k, _KH)

    acc = pl.pallas_call(
        functools.partial(_gfinal_body, n_total=float(n)),
        grid=(gg,),
        in_specs=[spec_rbg, spec_heTg, spec_hng, spec_he256g,
                  pl.BlockSpec((_HID, _KH), lambda i: (0, 0)),
                  pl.BlockSpec((_KH, _KH), lambda i: (0, 0)),
                  pl.BlockSpec((_KH, _KH), lambda i: (0, 0)),
                  pl.BlockSpec((_KH, _KH), lambda i: (0, 0)),
                  pl.BlockSpec((_KH, _KH), lambda i: (0, 0)),
                  wspec(fc1_w), pl.BlockSpec((1, _HID), lambda i: (0, 0)),
                  wspec(fc2_w), pl.BlockSpec((1, out_dim), lambda i: (0, 0))],
        out_specs=pl.BlockSpec((1, out_dim), lambda i: (0, 0)),
        out_shape=jax.ShapeDtypeStruct((1, out_dim), f32),
    )(r, he2, hn2, xj2,
      tile8(Wg_src[2]), bd(Wg_dst[2]), bd(Wg_e[2]), bd(Wg_v[2]),
      bd(Wg_bf[2]),
      fc1_w, fc1_b[None, :], fc2_w, fc2_b[None, :])

    return acc[0]
